# baseline (speedup 1.0000x reference)
import functools

import jax
import jax.numpy as jnp
from jax import lax
from jax.experimental import pallas as pl
from jax.experimental.pallas import tpu as pltpu
from jax.experimental.pallas import tpu_sc as plsc

N_ROWS = 2048
N_COLS = 8192
TOPK = 2458
L = 16
NW = 32
R_TC = 1344
R_SC = N_ROWS - R_TC
ROWS_PER_W = R_SC // NW
NV = N_COLS // L
UNROLL = 8
TC_BLOCK = 64

SKEY_NEG_INF = -2139095041
SKEY_POS_INF = 2139095040
BKEY_NEG_INF = 127
BKEY_POS_INF = 65408
NV16 = N_COLS // 128


def _skey_to_f32vec(k):
    kv = jnp.full((L,), k, dtype=jnp.int32)
    u = kv ^ (lax.shift_right_arithmetic(kv, 31) & jnp.int32(0x7FFFFFFF))
    return lax.bitcast_convert_type(u, jnp.float32)


def _lane_sum_i32(vec, scratch_ref):
    del scratch_ref
    total = vec[0]
    for i in range(1, L):
        total = total + vec[i]
    return total


def _sc_body(s_hbm, out_hbm, xbuf, obuf, cntbuf):
    cid = lax.axis_index("c")
    sid = lax.axis_index("s")
    wid = sid * 2 + cid
    row0 = wid * ROWS_PER_W

    def row_step(r, total_vec):
        pltpu.sync_copy(s_hbm.at[row0 + r], xbuf)

        def bis_step(_, state):
            lo, hi, tk, done = state
            mid = (lax.shift_right_arithmetic(lo, 1)
                   + lax.shift_right_arithmetic(hi, 1)
                   + (lo & hi & jnp.int32(1)))
            tvec = _skey_to_f32vec(mid)

            def cnt_step(i, acc):
                base = i * (UNROLL * L)
                for uu in range(UNROLL):
                    x = xbuf[pl.ds(base + uu * L, L)]
                    acc = acc + jnp.where(x >= tvec, jnp.int32(1), jnp.int32(0))
                return acc

            n_act = jnp.where(done, jnp.int32(0), jnp.int32(NV // UNROLL))
            cnt_vec = lax.fori_loop(0, n_act, cnt_step,
                                    jnp.zeros((L,), jnp.int32))
            cnt = _lane_sum_i32(cnt_vec, cntbuf)
            hit = jnp.logical_and(jnp.logical_not(done), cnt == TOPK)
            ge = cnt >= TOPK
            lo2 = jnp.where(jnp.logical_or(done, jnp.logical_not(ge)), lo, mid)
            hi2 = jnp.where(jnp.logical_or(done, ge), hi, mid)
            done2 = jnp.logical_or(done,
                                   jnp.logical_or(hit, hi2 - lo2 <= 1))
            tk2 = jnp.where(done, tk, jnp.where(hit, mid, lo2))
            return (lo2, hi2, tk2, done2)

        _, _, tk, _ = lax.fori_loop(
            0, 33, bis_step,
            (jnp.int32(SKEY_NEG_INF), jnp.int32(SKEY_POS_INF),
             jnp.int32(SKEY_NEG_INF), jnp.bool_(False)))
        tvec = _skey_to_f32vec(tk)

        def sum_step(i, carry):
            sacc, cacc = carry
            base = i * (UNROLL * L)
            for uu in range(UNROLL):
                x = xbuf[pl.ds(base + uu * L, L)]
                m = x > tvec
                sacc = sacc + jnp.where(m, x, jnp.float32(0))
                cacc = cacc + jnp.where(m, jnp.int32(1), jnp.int32(0))
            return (sacc, cacc)

        sacc, cacc = lax.fori_loop(0, NV // UNROLL, sum_step,
                                   (jnp.zeros((L,), jnp.float32),
                                    jnp.zeros((L,), jnp.int32)))
        cgt = _lane_sum_i32(cacc, cntbuf)
        corr_lane = (jnp.float32(TOPK) - cgt.astype(jnp.float32)) * jnp.float32(1.0 / L)
        return total_vec + sacc + jnp.full((L,), corr_lane, jnp.float32) * tvec

    total_vec = lax.fori_loop(0, ROWS_PER_W, row_step, jnp.zeros((L,), jnp.float32))
    obuf[...] = total_vec
    pltpu.sync_copy(obuf, out_hbm.at[wid])


_sc_topk = functools.partial(
    pl.kernel,
    out_type=jax.ShapeDtypeStruct((NW, L), jnp.float32),
    mesh=plsc.VectorSubcoreMesh(core_axis_name="c", subcore_axis_name="s"),
    scratch_types=[
        pltpu.VMEM((N_COLS,), jnp.float32),
        pltpu.VMEM((L,), jnp.float32),
        pltpu.VMEM((L,), jnp.int32),
    ],
)(_sc_body)


def _tc_inv_skey(kv):
    u = kv ^ (lax.shift_right_arithmetic(kv, 31) & jnp.int32(0x7FFFFFFF))
    return lax.bitcast_convert_type(u, jnp.float32)


def _tc_body(x_ref, o_ref):
    x = x_ref[...]

    lo32 = jnp.full((TC_BLOCK, 1), SKEY_NEG_INF, jnp.int32)
    hi32 = jnp.full((TC_BLOCK, 1), SKEY_POS_INF, jnp.int32)

    def bis_cond(state):
        _, _, _, done = state
        return jnp.min(done) == 0

    def bis(state):
        lo, hi, tk, done = state
        pend = done == 0
        mid = (lax.shift_right_arithmetic(lo, 1)
               + lax.shift_right_arithmetic(hi, 1)
               + (lo & hi & jnp.int32(1)))
        t = _tc_inv_skey(mid)
        cnt = jnp.sum((x >= t).astype(jnp.float32), axis=1, keepdims=True)
        hit = jnp.logical_and(pend, cnt == jnp.float32(TOPK))
        upd = jnp.logical_and(pend, cnt >= jnp.float32(TOPK))
        lo2 = jnp.where(upd, mid, lo)
        hi2 = jnp.where(jnp.logical_and(pend, jnp.logical_not(upd)), mid, hi)
        fin = jnp.logical_or(hit, hi2 - lo2 <= 1)
        done2 = jnp.where(jnp.logical_and(pend, fin), jnp.int32(1), done)
        tk2 = jnp.where(pend, jnp.where(hit, mid, lo2), tk)
        return (lo2, hi2, tk2, done2)

    _, _, tk, _ = lax.while_loop(
        bis_cond, bis,
        (lo32, hi32, lo32, jnp.zeros((TC_BLOCK, 1), jnp.int32)))
    t = _tc_inv_skey(tk)
    m = x > t
    ssum = jnp.sum(jnp.where(m, x, jnp.float32(0)), axis=1, keepdims=True)
    cgt = jnp.sum(m.astype(jnp.float32), axis=1, keepdims=True)
    o_ref[...] = ssum + (jnp.float32(TOPK) - cgt) * t


_tc_topk = pl.pallas_call(
    _tc_body,
    grid=(R_TC // TC_BLOCK,),
    in_specs=[pl.BlockSpec((TC_BLOCK, N_COLS),
                           lambda i: (R_SC // TC_BLOCK + i, 0))],
    out_specs=pl.BlockSpec((TC_BLOCK, 1), lambda i: (i, 0)),
    out_shape=jax.ShapeDtypeStruct((R_TC, 1), jnp.float32),
)


def _reduce_body(p_ref, q_ref, o_ref):
    total = ((jnp.sum(p_ref[...]) + jnp.sum(q_ref[...]))
             * jnp.float32(1.0 / (N_ROWS * TOPK)))
    o_ref[...] = jnp.full((1, 1), total, jnp.float32)


def kernel(s):
    partials = _sc_topk(s)
    tc_sums = _tc_topk(s)
    out = pl.pallas_call(
        _reduce_body,
        out_shape=jax.ShapeDtypeStruct((1, 1), jnp.float32),
    )(partials, tc_sums)
    return out[0, 0]


# speedup vs baseline: 1.1493x; 1.1493x over previous
import functools

import jax
import jax.numpy as jnp
from jax import lax
from jax.experimental import pallas as pl
from jax.experimental.pallas import tpu as pltpu
from jax.experimental.pallas import tpu_sc as plsc

N_ROWS = 2048
N_COLS = 8192
TOPK = 2458
L = 16
NW = 32
R_TC = 1408
R_SC = N_ROWS - R_TC
ROWS_PER_W = R_SC // NW
NV = N_COLS // L
UNROLL = 8
TC_BLOCK = 128

SKEY_NEG_INF = -2139095041
SKEY_POS_INF = 2139095040
BKEY_NEG_INF = 127
BKEY_POS_INF = 65408
NV16 = N_COLS // 128


def _skey_to_f32vec(k):
    kv = jnp.full((L,), k, dtype=jnp.int32)
    u = kv ^ (lax.shift_right_arithmetic(kv, 31) & jnp.int32(0x7FFFFFFF))
    return lax.bitcast_convert_type(u, jnp.float32)


def _lane_sum_i32(vec, scratch_ref):
    del scratch_ref
    total = vec[0]
    for i in range(1, L):
        total = total + vec[i]
    return total


def _sc_body(s_hbm, out_hbm, xbuf, obuf, cntbuf):
    cid = lax.axis_index("c")
    sid = lax.axis_index("s")
    wid = sid * 2 + cid
    row0 = wid * ROWS_PER_W

    def row_step(r, total_vec):
        pltpu.sync_copy(s_hbm.at[row0 + r], xbuf)

        def bis_step(_, state):
            lo, hi, tk, done = state
            mid = (lax.shift_right_arithmetic(lo, 1)
                   + lax.shift_right_arithmetic(hi, 1)
                   + (lo & hi & jnp.int32(1)))
            tvec = _skey_to_f32vec(mid)

            def cnt_step(i, acc):
                base = i * (UNROLL * L)
                for uu in range(UNROLL):
                    x = xbuf[pl.ds(base + uu * L, L)]
                    acc = acc + jnp.where(x >= tvec, jnp.int32(1), jnp.int32(0))
                return acc

            n_act = jnp.where(done, jnp.int32(0), jnp.int32(NV // UNROLL))
            cnt_vec = lax.fori_loop(0, n_act, cnt_step,
                                    jnp.zeros((L,), jnp.int32))
            cnt = _lane_sum_i32(cnt_vec, cntbuf)
            hit = jnp.logical_and(jnp.logical_not(done), cnt == TOPK)
            ge = cnt >= TOPK
            lo2 = jnp.where(jnp.logical_or(done, jnp.logical_not(ge)), lo, mid)
            hi2 = jnp.where(jnp.logical_or(done, ge), hi, mid)
            done2 = jnp.logical_or(done,
                                   jnp.logical_or(hit, hi2 - lo2 <= 1))
            tk2 = jnp.where(done, tk, jnp.where(hit, mid, lo2))
            return (lo2, hi2, tk2, done2)

        _, _, tk, _ = lax.fori_loop(
            0, 33, bis_step,
            (jnp.int32(SKEY_NEG_INF), jnp.int32(SKEY_POS_INF),
             jnp.int32(SKEY_NEG_INF), jnp.bool_(False)))
        tvec = _skey_to_f32vec(tk)

        def sum_step(i, carry):
            sacc, cacc = carry
            base = i * (UNROLL * L)
            for uu in range(UNROLL):
                x = xbuf[pl.ds(base + uu * L, L)]
                m = x > tvec
                sacc = sacc + jnp.where(m, x, jnp.float32(0))
                cacc = cacc + jnp.where(m, jnp.int32(1), jnp.int32(0))
            return (sacc, cacc)

        sacc, cacc = lax.fori_loop(0, NV // UNROLL, sum_step,
                                   (jnp.zeros((L,), jnp.float32),
                                    jnp.zeros((L,), jnp.int32)))
        cgt = _lane_sum_i32(cacc, cntbuf)
        corr_lane = (jnp.float32(TOPK) - cgt.astype(jnp.float32)) * jnp.float32(1.0 / L)
        return total_vec + sacc + jnp.full((L,), corr_lane, jnp.float32) * tvec

    total_vec = lax.fori_loop(0, ROWS_PER_W, row_step, jnp.zeros((L,), jnp.float32))
    obuf[...] = total_vec
    pltpu.sync_copy(obuf, out_hbm.at[wid])


_sc_topk = functools.partial(
    pl.kernel,
    out_type=jax.ShapeDtypeStruct((NW, L), jnp.float32),
    mesh=plsc.VectorSubcoreMesh(core_axis_name="c", subcore_axis_name="s"),
    scratch_types=[
        pltpu.VMEM((N_COLS,), jnp.float32),
        pltpu.VMEM((L,), jnp.float32),
        pltpu.VMEM((L,), jnp.int32),
    ],
)(_sc_body)


def _tc_inv_skey(kv):
    u = kv ^ (lax.shift_right_arithmetic(kv, 31) & jnp.int32(0x7FFFFFFF))
    return lax.bitcast_convert_type(u, jnp.float32)


def _tc_body(x_ref, o_ref):
    x = x_ref[...]

    lo32 = jnp.full((TC_BLOCK, 1), SKEY_NEG_INF, jnp.int32)
    hi32 = jnp.full((TC_BLOCK, 1), SKEY_POS_INF, jnp.int32)

    def bis_cond(state):
        _, _, _, done = state
        return jnp.min(done) == 0

    def bis(state):
        lo, hi, tk, done = state
        pend = done == 0
        mid = (lax.shift_right_arithmetic(lo, 1)
               + lax.shift_right_arithmetic(hi, 1)
               + (lo & hi & jnp.int32(1)))
        t = _tc_inv_skey(mid)
        cnt = jnp.sum((x >= t).astype(jnp.float32), axis=1, keepdims=True)
        hit = jnp.logical_and(pend, cnt == jnp.float32(TOPK))
        upd = jnp.logical_and(pend, cnt >= jnp.float32(TOPK))
        lo2 = jnp.where(upd, mid, lo)
        hi2 = jnp.where(jnp.logical_and(pend, jnp.logical_not(upd)), mid, hi)
        fin = jnp.logical_or(hit, hi2 - lo2 <= 1)
        done2 = jnp.where(jnp.logical_and(pend, fin), jnp.int32(1), done)
        tk2 = jnp.where(pend, jnp.where(hit, mid, lo2), tk)
        return (lo2, hi2, tk2, done2)

    _, _, tk, _ = lax.while_loop(
        bis_cond, bis,
        (lo32, hi32, lo32, jnp.zeros((TC_BLOCK, 1), jnp.int32)))
    t = _tc_inv_skey(tk)
    m = x > t
    ssum = jnp.sum(jnp.where(m, x, jnp.float32(0)), axis=1, keepdims=True)
    cgt = jnp.sum(m.astype(jnp.float32), axis=1, keepdims=True)
    o_ref[...] = ssum + (jnp.float32(TOPK) - cgt) * t


_tc_topk = pl.pallas_call(
    _tc_body,
    grid=(R_TC // TC_BLOCK,),
    in_specs=[pl.BlockSpec((TC_BLOCK, N_COLS),
                           lambda i: (R_SC // TC_BLOCK + i, 0))],
    out_specs=pl.BlockSpec((TC_BLOCK, 1), lambda i: (i, 0)),
    out_shape=jax.ShapeDtypeStruct((R_TC, 1), jnp.float32),
)


def _reduce_body(p_ref, q_ref, o_ref):
    total = ((jnp.sum(p_ref[...]) + jnp.sum(q_ref[...]))
             * jnp.float32(1.0 / (N_ROWS * TOPK)))
    o_ref[...] = jnp.full((1, 1), total, jnp.float32)


def kernel(s):
    partials = _sc_topk(s)
    tc_sums = _tc_topk(s)
    out = pl.pallas_call(
        _reduce_body,
        out_shape=jax.ShapeDtypeStruct((1, 1), jnp.float32),
    )(partials, tc_sums)
    return out[0, 0]
